# baseline (speedup 1.0000x reference)
import jax
import jax.numpy as jnp
from jax import lax
from jax.experimental import pallas as pl
from jax.experimental.pallas import tpu as pltpu
from jax.experimental.pallas import tpu_sc as plsc

NREL = 1000
B = 1024

NC = 2
NS = 16
NW = NC * NS
BPW = B // NW

G = 3
SZS = [G] * (BPW // G) + ([BPW % G] if BPW % G else [])
OFFS = [G * i for i in range(len(SZS))]
NCH = len(SZS)


def _weight_body(relp_hbm, w_hbm, w_out,
                 idxp, wbufs, gsems, wsems):
    cid = lax.axis_index("c")
    sid = lax.axis_index("s")
    wid = sid * NC + cid
    base = wid * BPW

    pltpu.sync_copy(relp_hbm.at[wid], idxp)

    def gather(k):
        return pltpu.async_copy(
            w_hbm.at[idxp.at[k, pl.ds(0, SZS[k])]],
            wbufs[k % 2].at[pl.ds(0, SZS[k])], gsems[k % 2])

    gath = [gather(0), gather(1)]
    wrs = [None, None]
    for j in range(NCH):
        b = j % 2
        gath[b].wait()
        wrs[b] = pltpu.async_copy(
            wbufs[b].at[pl.ds(0, SZS[j])],
            w_out.at[pl.ds(base + OFFS[j], SZS[j])], wsems[b])
        k = j + 2
        if k < NCH:
            wrs[b].wait()
            gath[b] = gather(k)

    wrs[(NCH - 2) % 2].wait()
    wrs[(NCH - 1) % 2].wait()


def _bias_body(idx_ref, b_ref, out_ref):
    def body(i, carry):
        out_ref[pl.ds(i, 1)] = b_ref[pl.ds(idx_ref[i], 1)]
        return carry
    lax.fori_loop(0, B, body, 0)


@jax.jit
def kernel(relation, mlp_weight, mlp_bias):
    pos = jnp.minimum(
        jnp.array(OFFS, jnp.int32)[:, None] + jnp.arange(8, dtype=jnp.int32),
        BPW - 1)
    relp = relation.reshape(NW, BPW)[:, pos]

    wk = pl.kernel(
        _weight_body,
        out_type=jax.ShapeDtypeStruct((B, 128, 128), jnp.float32),
        mesh=plsc.VectorSubcoreMesh(core_axis_name="c", subcore_axis_name="s"),
        scratch_types=[
            pltpu.VMEM((NCH, 8), jnp.int32),
            tuple(pltpu.VMEM((G, 128, 128), jnp.float32) for _ in range(2)),
            tuple(pltpu.SemaphoreType.DMA for _ in range(2)),
            tuple(pltpu.SemaphoreType.DMA for _ in range(2)),
        ],
    )
    w_out = wk(relp, mlp_weight)

    b_out = pl.pallas_call(
        _bias_body,
        grid_spec=pltpu.PrefetchScalarGridSpec(
            num_scalar_prefetch=1,
            grid=(1,),
            in_specs=[
                pl.BlockSpec((NREL, 8, 128), lambda i, idx: (0, 0, 0)),
            ],
            out_specs=pl.BlockSpec((B, 8, 128), lambda i, idx: (0, 0, 0)),
        ),
        out_shape=jax.ShapeDtypeStruct((B, 8, 128), jnp.float32),
    )(relation, mlp_bias)

    return w_out, b_out
